# baseline (speedup 1.0000x reference)
import functools

import jax
import jax.numpy as jnp
from jax import lax
from jax.experimental import pallas as pl
from jax.experimental.pallas import tpu as pltpu
from jax.experimental.pallas import tpu_sc as plsc

N = 10000
NPAD = 10240
E = 640000
D_IN = 1433
HID = 128
HEADS = 2
NCLS = 7
GW = 16

NWORK = 32
CH = 128
NCHUNK = E // CH
CPW = NCHUNK // NWORK
EXTRA = NCHUNK - CPW * NWORK
K = 4
NSUP = CPW // K
KP = 2
NSUPP = CPW // KP
BM = 256

F32 = jnp.float32


def _mesh():
    return plsc.VectorSubcoreMesh(
        core_axis_name="c", subcore_axis_name="s", num_cores=2, num_subcores=16
    )


_SC_PARAMS = pltpu.CompilerParams(needs_layout_passes=False, use_tc_tiling_on_sc=False)


def _zero_1d(ref, n):
    z = jnp.zeros((16,), F32)

    def row(i, _):
        ref[pl.ds(i * 16, 16)] = z
        return 0

    lax.fori_loop(0, n // 16, row, 0)


def _zero_2d(ref, rows, cols):
    z = jnp.zeros((16,), F32)

    def row(i, _):
        for j in range(cols // 16):
            ref[i, pl.ds(j * 16, 16)] = z
        return 0

    lax.fori_loop(0, rows, row, 0)


def _sc_deg(dst2d):
    @functools.partial(
        pl.kernel,
        out_type=jax.ShapeDtypeStruct((NWORK * NPAD,), F32),
        mesh=_mesh(),
        compiler_params=_SC_PARAMS,
        scratch_types=[
            pltpu.VMEM((NPAD,), F32),
            pltpu.VMEM((K, CH), jnp.int32),
        ],
    )
    def run(dst_hbm, out_hbm, hist, dbuf):
        c = lax.axis_index("c")
        s = lax.axis_index("s")
        wid = c * 16 + s
        _zero_1d(hist, NPAD)
        ones16 = jnp.ones((16,), F32)

        def sup(g, _):
            base = wid * CPW + g * K
            pltpu.sync_copy(dst_hbm.at[pl.ds(base, K)], dbuf)

            @plsc.parallel_loop(0, K * CH // 16, unroll=4)
            def hloop(q):
                idx = dbuf[q // 8, pl.ds((q % 8) * 16, 16)]
                plsc.addupdate_scatter(hist, [idx], ones16)
            return 0

        lax.fori_loop(0, NSUP, sup, 0)

        @pl.when(wid < EXTRA)
        def _():
            ci = NCHUNK - EXTRA + wid
            pltpu.sync_copy(dst_hbm.at[pl.ds(ci, 1)], dbuf.at[pl.ds(0, 1)])
            for q in range(CH // 16):
                idx = dbuf[0, pl.ds(q * 16, 16)]
                plsc.addupdate_scatter(hist, [idx], ones16)

        pltpu.sync_copy(hist, out_hbm.at[pl.ds(wid * NPAD, NPAD)])

    return run(dst2d)


def _sc_prop(hs, src2d, dst2d):
    @functools.partial(
        pl.kernel,
        out_type=jax.ShapeDtypeStruct((2 * NPAD, HID), F32),
        mesh=_mesh(),
        compiler_params=_SC_PARAMS,
        scratch_types=[
            pltpu.VMEM_SHARED((NPAD, HID), F32),
            pltpu.VMEM((KP, CH, HID), F32),
            pltpu.VMEM((KP, CH), jnp.int32),
            pltpu.VMEM((KP, CH), jnp.int32),
            pltpu.SemaphoreType.DMA,
            pltpu.SemaphoreType.DMA,
        ],
    )
    def run(hs_hbm, src_hbm, dst_hbm, out_hbm, acc, rows, sbuf, dbuf, gsem, ssem):
        c = lax.axis_index("c")
        s = lax.axis_index("s")
        wid = c * 16 + s

        _zero_2d(rows.at[0], CH, HID)
        for t in range(5):
            pltpu.sync_copy(rows.at[0], acc.at[pl.ds(s * 640 + t * CH, CH)])
        plsc.subcore_barrier()

        def sup(g, _):
            base = wid * CPW + g * KP
            pltpu.sync_copy(src_hbm.at[pl.ds(base, KP)], sbuf)
            pltpu.sync_copy(dst_hbm.at[pl.ds(base, KP)], dbuf)
            gets = [
                pltpu.async_copy(hs_hbm.at[sbuf.at[k]], rows.at[k], gsem)
                for k in range(KP)
            ]
            puts = []
            for k in range(KP):
                gets[k].wait()
                puts.append(
                    pltpu.async_copy(rows.at[k], acc.at[dbuf.at[k]], ssem, add=True)
                )
            for d in puts:
                d.wait()
            return 0

        lax.fori_loop(0, NSUPP, sup, 0)

        @pl.when(wid < EXTRA)
        def _():
            ci = NCHUNK - EXTRA + wid
            pltpu.sync_copy(src_hbm.at[pl.ds(ci, 1)], sbuf.at[pl.ds(0, 1)])
            pltpu.sync_copy(dst_hbm.at[pl.ds(ci, 1)], dbuf.at[pl.ds(0, 1)])
            pltpu.async_copy(hs_hbm.at[sbuf.at[0]], rows.at[0], gsem).wait()
            pltpu.async_copy(rows.at[0], acc.at[dbuf.at[0]], ssem, add=True).wait()

        plsc.subcore_barrier()
        for t in range(5):
            r0 = s * 640 + t * CH
            pltpu.sync_copy(acc.at[pl.ds(r0, CH)], rows.at[0])
            pltpu.sync_copy(rows.at[0], out_hbm.at[pl.ds(c * NPAD + r0, CH)])

    return run(hs, src2d, dst2d)


def _sc_gat(gtab, src2d, dst2d, as0, as1, ad0, ad1, am0, am1):
    @functools.partial(
        pl.kernel,
        out_type=jax.ShapeDtypeStruct((2 * NPAD, GW), F32),
        mesh=_mesh(),
        compiler_params=_SC_PARAMS,
        scratch_types=[
            pltpu.VMEM_SHARED((NPAD, GW), F32),
            pltpu.VMEM((K, CH, GW), F32),
            pltpu.VMEM((K, CH, GW), F32),
            pltpu.VMEM((K, CH), jnp.int32),
            pltpu.VMEM((K, CH), jnp.int32),
            pltpu.VMEM((N,), F32),
            pltpu.VMEM((N,), F32),
            pltpu.VMEM((N,), F32),
            pltpu.VMEM((N,), F32),
            pltpu.VMEM((16,), F32),
            pltpu.VMEM((16,), F32),
            pltpu.SemaphoreType.DMA,
            pltpu.SemaphoreType.DMA,
        ],
    )
    def run(g_hbm, src_hbm, dst_hbm, as0_hbm, as1_hbm, ad0_hbm, ad1_hbm,
            am0_hbm, am1_hbm, out_hbm,
            acc, grows, obuf, sbuf, dbuf, ast0, ast1, adt0, adt1, am0v, am1v,
            gsem, ssem):
        c = lax.axis_index("c")
        s = lax.axis_index("s")
        wid = c * 16 + s

        pltpu.sync_copy(as0_hbm, ast0)
        pltpu.sync_copy(as1_hbm, ast1)
        pltpu.sync_copy(ad0_hbm, adt0)
        pltpu.sync_copy(ad1_hbm, adt1)
        pltpu.sync_copy(am0_hbm, am0v)
        pltpu.sync_copy(am1_hbm, am1v)

        _zero_2d(grows.at[0], CH, GW)
        for t in range(5):
            pltpu.sync_copy(grows.at[0], acc.at[pl.ds(s * 640 + t * CH, CH)])
        plsc.subcore_barrier()

        iota16 = lax.iota(jnp.int32, 16)
        av0 = am0v[...]
        av1 = am1v[...]

        def do_chunk(k):
            @plsc.parallel_loop(0, CH // 16, unroll=4)
            def group(q):
                sv = sbuf[k, pl.ds(q * 16, 16)]
                dv = dbuf[k, pl.ds(q * 16, 16)]
                vas0 = plsc.load_gather(ast0, [sv])
                vas1 = plsc.load_gather(ast1, [sv])
                vad0 = plsc.load_gather(adt0, [dv])
                vad1 = plsc.load_gather(adt1, [dv])
                t0 = av0 + vad0
                m0 = jnp.maximum(t0, 0.2 * t0)
                t1 = av1 + vad1
                m1 = jnp.maximum(t1, 0.2 * t1)
                e0 = vas0 + vad0
                e0 = jnp.maximum(e0, 0.2 * e0)
                p0 = jnp.exp(e0 - m0)
                e1 = vas1 + vad1
                e1 = jnp.maximum(e1, 0.2 * e1)
                p1 = jnp.exp(e1 - m1)
                kv = jnp.full((16,), k, jnp.int32)
                rv = q * 16 + iota16
                for j in range(GW):
                    jv = jnp.full((16,), j, jnp.int32)
                    v = plsc.load_gather(grows, [kv, rv, jv])
                    v = v * (p0 if j < 8 else p1)
                    plsc.store_scatter(obuf, [kv, rv, jv], v)

        def sup(g, _):
            base = wid * CPW + g * K
            pltpu.sync_copy(src_hbm.at[pl.ds(base, K)], sbuf)
            pltpu.sync_copy(dst_hbm.at[pl.ds(base, K)], dbuf)
            gets = [
                pltpu.async_copy(g_hbm.at[sbuf.at[k]], grows.at[k], gsem)
                for k in range(K)
            ]
            puts = []
            for k in range(K):
                gets[k].wait()
                do_chunk(k)
                puts.append(
                    pltpu.async_copy(obuf.at[k], acc.at[dbuf.at[k]], ssem, add=True)
                )
            for d in puts:
                d.wait()
            return 0

        lax.fori_loop(0, NSUP, sup, 0)

        @pl.when(wid < EXTRA)
        def _():
            ci = NCHUNK - EXTRA + wid
            pltpu.sync_copy(src_hbm.at[pl.ds(ci, 1)], sbuf.at[pl.ds(0, 1)])
            pltpu.sync_copy(dst_hbm.at[pl.ds(ci, 1)], dbuf.at[pl.ds(0, 1)])
            pltpu.async_copy(g_hbm.at[sbuf.at[0]], grows.at[0], gsem).wait()
            do_chunk(0)
            pltpu.async_copy(obuf.at[0], acc.at[dbuf.at[0]], ssem, add=True).wait()

        plsc.subcore_barrier()
        for t in range(5):
            r0 = s * 640 + t * CH
            pltpu.sync_copy(acc.at[pl.ds(r0, CH)], grows.at[0])
            pltpu.sync_copy(grows.at[0], out_hbm.at[pl.ds(c * NPAD + r0, CH)])

    return run(gtab, src2d, dst2d, as0, as1, ad0, ad1, am0, am1)


def _tc_dinv(degp):
    def body(d_ref, o_ref):
        deg = jnp.sum(d_ref[...], axis=0, keepdims=True) + 1.0
        o_ref[...] = lax.rsqrt(deg)

    return pl.pallas_call(
        body, out_shape=jax.ShapeDtypeStruct((1, NPAD), F32)
    )(degp)


def _tc_mm1(x, W1, dinv):
    def body(x_ref, w_ref, d_ref, o_ref):
        h = jnp.dot(x_ref[...], w_ref[...], preferred_element_type=F32)
        o_ref[...] = h * d_ref[...]

    return pl.pallas_call(
        body,
        grid=(NPAD // BM,),
        in_specs=[
            pl.BlockSpec((BM, D_IN), lambda i: (i, 0)),
            pl.BlockSpec((D_IN, HID), lambda i: (0, 0)),
            pl.BlockSpec((BM, 1), lambda i: (i, 0)),
        ],
        out_specs=pl.BlockSpec((BM, HID), lambda i: (i, 0)),
        out_shape=jax.ShapeDtypeStruct((NPAD, HID), F32),
    )(x, W1, dinv)


def _tc_mid(p, hs, dinv, b, W):
    def body(p_ref, h_ref, d_ref, b_ref, w_ref, o_ref):
        agg = p_ref[0] + p_ref[1] + h_ref[...]
        z = jnp.maximum(agg * d_ref[...] + b_ref[...], 0.0)
        o_ref[...] = jnp.dot(z, w_ref[...], preferred_element_type=F32) * d_ref[...]

    return pl.pallas_call(
        body,
        grid=(NPAD // BM,),
        in_specs=[
            pl.BlockSpec((2, BM, HID), lambda i: (0, i, 0)),
            pl.BlockSpec((BM, HID), lambda i: (i, 0)),
            pl.BlockSpec((BM, 1), lambda i: (i, 0)),
            pl.BlockSpec((1, HID), lambda i: (0, 0)),
            pl.BlockSpec((HID, HID), lambda i: (0, 0)),
        ],
        out_specs=pl.BlockSpec((BM, HID), lambda i: (i, 0)),
        out_shape=jax.ShapeDtypeStruct((NPAD, HID), F32),
    )(p, hs, dinv, b, W)


def _tc_gat_prep(p, hs, dinv, b, W3, att_src, att_dst):
    def body(p_ref, h_ref, d_ref, b_ref, w_ref, asr, adr, g_ref, as_ref, ad_ref):
        agg = p_ref[0] + p_ref[1] + h_ref[...]
        z = jnp.maximum(agg * d_ref[...] + b_ref[...], 0.0)
        h3 = jnp.dot(z, w_ref[...], preferred_element_type=F32)
        one = jnp.ones((BM, 1), F32)
        g_ref[...] = jnp.concatenate(
            [h3[:, 0:7], one, h3[:, 7:14], one], axis=1
        )
        va = asr[...]
        as0 = jnp.sum(h3[:, 0:7] * va[0:1, :], axis=1, keepdims=True)
        as1 = jnp.sum(h3[:, 7:14] * va[1:2, :], axis=1, keepdims=True)
        as_ref[...] = jnp.concatenate([as0, as1], axis=1)
        vd = adr[...]
        ad0 = jnp.sum(h3[:, 0:7] * vd[0:1, :], axis=1, keepdims=True)
        ad1 = jnp.sum(h3[:, 7:14] * vd[1:2, :], axis=1, keepdims=True)
        ad_ref[...] = jnp.concatenate([ad0, ad1], axis=1)

    return pl.pallas_call(
        body,
        grid=(NPAD // BM,),
        in_specs=[
            pl.BlockSpec((2, BM, HID), lambda i: (0, i, 0)),
            pl.BlockSpec((BM, HID), lambda i: (i, 0)),
            pl.BlockSpec((BM, 1), lambda i: (i, 0)),
            pl.BlockSpec((1, HID), lambda i: (0, 0)),
            pl.BlockSpec((HID, HEADS * NCLS), lambda i: (0, 0)),
            pl.BlockSpec((HEADS, NCLS), lambda i: (0, 0)),
            pl.BlockSpec((HEADS, NCLS), lambda i: (0, 0)),
        ],
        out_specs=[
            pl.BlockSpec((BM, GW), lambda i: (i, 0)),
            pl.BlockSpec((BM, 2), lambda i: (i, 0)),
            pl.BlockSpec((BM, 2), lambda i: (i, 0)),
        ],
        out_shape=[
            jax.ShapeDtypeStruct((N, GW), F32),
            jax.ShapeDtypeStruct((N, 2), F32),
            jax.ShapeDtypeStruct((N, 2), F32),
        ],
    )(p, hs, dinv, b, W3, att_src, att_dst)


def _tc_m(a_s, a_d):
    def body(as_ref, ad_ref, m_ref, am_ref):
        av = as_ref[...]
        amax = jnp.max(av, axis=0, keepdims=True)
        t = amax + ad_ref[...]
        m_ref[...] = jnp.maximum(t, 0.2 * t)
        am_ref[...] = jnp.broadcast_to(
            jnp.transpose(amax, (1, 0)), (HEADS, 16)
        )

    return pl.pallas_call(
        body,
        out_shape=[
            jax.ShapeDtypeStruct((N, 2), F32),
            jax.ShapeDtypeStruct((HEADS, 16), F32),
        ],
    )(a_s, a_d)


def _tc_final(pg, g, a_s, a_d, m, b3):
    def body(p_ref, g_ref, as_ref, ad_ref, m_ref, b_ref, o_ref):
        num = p_ref[0, :N, :] + p_ref[1, :N, :]
        es = as_ref[...] + ad_ref[...]
        es = jnp.maximum(es, 0.2 * es)
        ps = jnp.exp(es - m_ref[...])
        w = jnp.concatenate(
            [jnp.broadcast_to(ps[:, 0:1], (N, 8)),
             jnp.broadcast_to(ps[:, 1:2], (N, 8))], axis=1
        )
        num = num + g_ref[...] * w
        d0 = num[:, 7:8] + 1e-16
        d1 = num[:, 15:16] + 1e-16
        o = jnp.concatenate(
            [num[:, 0:7] / d0, num[:, 8:15] / d1], axis=1
        ) + b_ref[...]
        mx = jnp.max(o, axis=1, keepdims=True)
        t = o - mx
        o_ref[...] = t - jnp.log(jnp.sum(jnp.exp(t), axis=1, keepdims=True))

    return pl.pallas_call(
        body, out_shape=jax.ShapeDtypeStruct((N, HEADS * NCLS), F32)
    )(pg, g, a_s, a_d, m, b3)


def kernel(x, edge_index, W1, b1, W2, b2, W3, att_src, att_dst, b3):
    src2d = edge_index[0].reshape(NCHUNK, CH)
    dst2d = edge_index[1].reshape(NCHUNK, CH)

    degp = _sc_deg(dst2d).reshape(NWORK, NPAD)
    dinv = _tc_dinv(degp).reshape(NPAD, 1)

    hs1 = _tc_mm1(x, W1, dinv)
    p1 = _sc_prop(hs1, src2d, dst2d).reshape(2, NPAD, HID)
    hs2 = _tc_mid(p1, hs1, dinv, b1.reshape(1, HID), W2)
    p2 = _sc_prop(hs2, src2d, dst2d).reshape(2, NPAD, HID)
    g, a_s, a_d = _tc_gat_prep(
        p2, hs2, dinv, b2.reshape(1, HID), W3, att_src, att_dst
    )
    m, am = _tc_m(a_s, a_d)
    pg = _sc_gat(
        g, src2d, dst2d,
        a_s[:, 0], a_s[:, 1], a_d[:, 0], a_d[:, 1], am[0], am[1]
    ).reshape(2, NPAD, GW)
    return _tc_final(pg, g, a_s, a_d, m, b3.reshape(1, HEADS * NCLS))


# speedup vs baseline: 1.0168x; 1.0168x over previous
import functools

import jax
import jax.numpy as jnp
from jax import lax
from jax.experimental import pallas as pl
from jax.experimental.pallas import tpu as pltpu
from jax.experimental.pallas import tpu_sc as plsc

N = 10000
NPAD = 10240
E = 640000
D_IN = 1433
HID = 128
HEADS = 2
NCLS = 7
GW = 16

NWORK = 32
CH = 128
NCHUNK = E // CH
CPW = NCHUNK // NWORK
EXTRA = NCHUNK - CPW * NWORK
K = 4
NSUP = CPW // K
KP = 2
NSUPP = CPW // KP
BM = 256

F32 = jnp.float32


def _mesh():
    return plsc.VectorSubcoreMesh(
        core_axis_name="c", subcore_axis_name="s", num_cores=2, num_subcores=16
    )


_SC_PARAMS = pltpu.CompilerParams(needs_layout_passes=False, use_tc_tiling_on_sc=False)


def _zero_1d(ref, n):
    z = jnp.zeros((16,), F32)

    def row(i, _):
        ref[pl.ds(i * 16, 16)] = z
        return 0

    lax.fori_loop(0, n // 16, row, 0)


def _zero_2d(ref, rows, cols):
    z = jnp.zeros((16,), F32)

    def row(i, _):
        for j in range(cols // 16):
            ref[i, pl.ds(j * 16, 16)] = z
        return 0

    lax.fori_loop(0, rows, row, 0)


def _sc_deg(dst2d):
    @functools.partial(
        pl.kernel,
        out_type=jax.ShapeDtypeStruct((NWORK * NPAD,), F32),
        mesh=_mesh(),
        compiler_params=_SC_PARAMS,
        scratch_types=[
            pltpu.VMEM((NPAD,), F32),
            pltpu.VMEM((K, CH), jnp.int32),
        ],
    )
    def run(dst_hbm, out_hbm, hist, dbuf):
        c = lax.axis_index("c")
        s = lax.axis_index("s")
        wid = c * 16 + s
        _zero_1d(hist, NPAD)
        ones16 = jnp.ones((16,), F32)

        def sup(g, _):
            base = wid * CPW + g * K
            pltpu.sync_copy(dst_hbm.at[pl.ds(base, K)], dbuf)

            @plsc.parallel_loop(0, K * CH // 16, unroll=4)
            def hloop(q):
                idx = dbuf[q // 8, pl.ds((q % 8) * 16, 16)]
                plsc.addupdate_scatter(hist, [idx], ones16)
            return 0

        lax.fori_loop(0, NSUP, sup, 0)

        @pl.when(wid < EXTRA)
        def _():
            ci = NCHUNK - EXTRA + wid
            pltpu.sync_copy(dst_hbm.at[pl.ds(ci, 1)], dbuf.at[pl.ds(0, 1)])
            for q in range(CH // 16):
                idx = dbuf[0, pl.ds(q * 16, 16)]
                plsc.addupdate_scatter(hist, [idx], ones16)

        pltpu.sync_copy(hist, out_hbm.at[pl.ds(wid * NPAD, NPAD)])

    return run(dst2d)


def _sc_prop(hs, src2d, dst2d):
    @functools.partial(
        pl.kernel,
        out_type=jax.ShapeDtypeStruct((2 * NPAD, HID), F32),
        mesh=_mesh(),
        compiler_params=_SC_PARAMS,
        scratch_types=[
            pltpu.VMEM_SHARED((NPAD, HID), F32),
            pltpu.VMEM((KP, CH, HID), F32),
            pltpu.VMEM((KP, CH), jnp.int32),
            pltpu.VMEM((KP, CH), jnp.int32),
            pltpu.SemaphoreType.DMA,
            pltpu.SemaphoreType.DMA,
        ],
    )
    def run(hs_hbm, src_hbm, dst_hbm, out_hbm, acc, rows, sbuf, dbuf, gsem, ssem):
        c = lax.axis_index("c")
        s = lax.axis_index("s")
        wid = c * 16 + s

        _zero_2d(rows.at[0], CH, HID)
        for t in range(5):
            pltpu.sync_copy(rows.at[0], acc.at[pl.ds(s * 640 + t * CH, CH)])
        plsc.subcore_barrier()

        def sup(g, _):
            base = wid * CPW + g * KP
            pltpu.sync_copy(src_hbm.at[pl.ds(base, KP)], sbuf)
            pltpu.sync_copy(dst_hbm.at[pl.ds(base, KP)], dbuf)
            gets = [
                pltpu.async_copy(hs_hbm.at[sbuf.at[k]], rows.at[k], gsem)
                for k in range(KP)
            ]
            puts = []
            for k in range(KP):
                gets[k].wait()
                puts.append(
                    pltpu.async_copy(rows.at[k], acc.at[dbuf.at[k]], ssem, add=True)
                )
            for d in puts:
                d.wait()
            return 0

        lax.fori_loop(0, NSUPP, sup, 0)

        @pl.when(wid < EXTRA)
        def _():
            ci = NCHUNK - EXTRA + wid
            pltpu.sync_copy(src_hbm.at[pl.ds(ci, 1)], sbuf.at[pl.ds(0, 1)])
            pltpu.sync_copy(dst_hbm.at[pl.ds(ci, 1)], dbuf.at[pl.ds(0, 1)])
            pltpu.async_copy(hs_hbm.at[sbuf.at[0]], rows.at[0], gsem).wait()
            pltpu.async_copy(rows.at[0], acc.at[dbuf.at[0]], ssem, add=True).wait()

        plsc.subcore_barrier()
        for t in range(5):
            r0 = s * 640 + t * CH
            pltpu.sync_copy(acc.at[pl.ds(r0, CH)], rows.at[0])
            pltpu.sync_copy(rows.at[0], out_hbm.at[pl.ds(c * NPAD + r0, CH)])

    return run(hs, src2d, dst2d)


def _sc_gat(gtab, src2d, dst2d, as0, as1, ad0, ad1, am0, am1):
    @functools.partial(
        pl.kernel,
        out_type=jax.ShapeDtypeStruct((2 * NPAD, GW), F32),
        mesh=_mesh(),
        compiler_params=_SC_PARAMS,
        scratch_types=[
            pltpu.VMEM_SHARED((NPAD, GW), F32),
            pltpu.VMEM((K, CH, GW), F32),
            pltpu.VMEM((K, CH, GW), F32),
            pltpu.VMEM((K, CH), jnp.int32),
            pltpu.VMEM((K, CH), jnp.int32),
            pltpu.VMEM((N,), F32),
            pltpu.VMEM((N,), F32),
            pltpu.VMEM((N,), F32),
            pltpu.VMEM((N,), F32),
            pltpu.VMEM((16,), F32),
            pltpu.VMEM((16,), F32),
            pltpu.SemaphoreType.DMA,
            pltpu.SemaphoreType.DMA,
        ],
    )
    def run(g_hbm, src_hbm, dst_hbm, as0_hbm, as1_hbm, ad0_hbm, ad1_hbm,
            am0_hbm, am1_hbm, out_hbm,
            acc, grows, obuf, sbuf, dbuf, ast0, ast1, adt0, adt1, am0v, am1v,
            gsem, ssem):
        c = lax.axis_index("c")
        s = lax.axis_index("s")
        wid = c * 16 + s

        pltpu.sync_copy(as0_hbm, ast0)
        pltpu.sync_copy(as1_hbm, ast1)
        pltpu.sync_copy(ad0_hbm, adt0)
        pltpu.sync_copy(ad1_hbm, adt1)
        pltpu.sync_copy(am0_hbm, am0v)
        pltpu.sync_copy(am1_hbm, am1v)

        _zero_2d(grows.at[0], CH, GW)
        for t in range(5):
            pltpu.sync_copy(grows.at[0], acc.at[pl.ds(s * 640 + t * CH, CH)])
        plsc.subcore_barrier()

        iota16 = lax.iota(jnp.int32, 16)
        av0 = am0v[...]
        av1 = am1v[...]

        def do_chunk(k):
            @plsc.parallel_loop(0, CH // 16, unroll=2)
            def group(q):
                sv = sbuf[k, pl.ds(q * 16, 16)]
                dv = dbuf[k, pl.ds(q * 16, 16)]
                vas0 = plsc.load_gather(ast0, [sv])
                vas1 = plsc.load_gather(ast1, [sv])
                vad0 = plsc.load_gather(adt0, [dv])
                vad1 = plsc.load_gather(adt1, [dv])
                t0 = av0 + vad0
                m0 = jnp.maximum(t0, 0.2 * t0)
                t1 = av1 + vad1
                m1 = jnp.maximum(t1, 0.2 * t1)
                e0 = vas0 + vad0
                e0 = jnp.maximum(e0, 0.2 * e0)
                p0 = jnp.exp(e0 - m0)
                e1 = vas1 + vad1
                e1 = jnp.maximum(e1, 0.2 * e1)
                p1 = jnp.exp(e1 - m1)
                kv = jnp.full((16,), k, jnp.int32)
                rv = q * 16 + iota16
                for j in range(GW):
                    jv = jnp.full((16,), j, jnp.int32)
                    v = plsc.load_gather(grows, [kv, rv, jv])
                    v = v * (p0 if j < 8 else p1)
                    plsc.store_scatter(obuf, [kv, rv, jv], v)

        def sup(g, _):
            base = wid * CPW + g * K
            pltpu.sync_copy(src_hbm.at[pl.ds(base, K)], sbuf)
            pltpu.sync_copy(dst_hbm.at[pl.ds(base, K)], dbuf)
            gets = [
                pltpu.async_copy(g_hbm.at[sbuf.at[k]], grows.at[k], gsem)
                for k in range(K)
            ]
            puts = []
            for k in range(K):
                gets[k].wait()
                do_chunk(k)
                puts.append(
                    pltpu.async_copy(obuf.at[k], acc.at[dbuf.at[k]], ssem, add=True)
                )
            for d in puts:
                d.wait()
            return 0

        lax.fori_loop(0, NSUP, sup, 0)

        @pl.when(wid < EXTRA)
        def _():
            ci = NCHUNK - EXTRA + wid
            pltpu.sync_copy(src_hbm.at[pl.ds(ci, 1)], sbuf.at[pl.ds(0, 1)])
            pltpu.sync_copy(dst_hbm.at[pl.ds(ci, 1)], dbuf.at[pl.ds(0, 1)])
            pltpu.async_copy(g_hbm.at[sbuf.at[0]], grows.at[0], gsem).wait()
            do_chunk(0)
            pltpu.async_copy(obuf.at[0], acc.at[dbuf.at[0]], ssem, add=True).wait()

        plsc.subcore_barrier()
        for t in range(5):
            r0 = s * 640 + t * CH
            pltpu.sync_copy(acc.at[pl.ds(r0, CH)], grows.at[0])
            pltpu.sync_copy(grows.at[0], out_hbm.at[pl.ds(c * NPAD + r0, CH)])

    return run(gtab, src2d, dst2d, as0, as1, ad0, ad1, am0, am1)


def _tc_dinv(degp):
    def body(d_ref, o_ref):
        deg = jnp.sum(d_ref[...], axis=0, keepdims=True) + 1.0
        o_ref[...] = lax.rsqrt(deg)

    return pl.pallas_call(
        body, out_shape=jax.ShapeDtypeStruct((1, NPAD), F32)
    )(degp)


def _tc_mm1(x, W1, dinv):
    def body(x_ref, w_ref, d_ref, o_ref):
        h = jnp.dot(x_ref[...], w_ref[...], preferred_element_type=F32)
        o_ref[...] = h * d_ref[...]

    return pl.pallas_call(
        body,
        grid=(NPAD // BM,),
        in_specs=[
            pl.BlockSpec((BM, D_IN), lambda i: (i, 0)),
            pl.BlockSpec((D_IN, HID), lambda i: (0, 0)),
            pl.BlockSpec((BM, 1), lambda i: (i, 0)),
        ],
        out_specs=pl.BlockSpec((BM, HID), lambda i: (i, 0)),
        out_shape=jax.ShapeDtypeStruct((NPAD, HID), F32),
    )(x, W1, dinv)


def _tc_mid(p, hs, dinv, b, W):
    def body(p_ref, h_ref, d_ref, b_ref, w_ref, o_ref):
        agg = p_ref[0] + p_ref[1] + h_ref[...]
        z = jnp.maximum(agg * d_ref[...] + b_ref[...], 0.0)
        o_ref[...] = jnp.dot(z, w_ref[...], preferred_element_type=F32) * d_ref[...]

    return pl.pallas_call(
        body,
        grid=(NPAD // BM,),
        in_specs=[
            pl.BlockSpec((2, BM, HID), lambda i: (0, i, 0)),
            pl.BlockSpec((BM, HID), lambda i: (i, 0)),
            pl.BlockSpec((BM, 1), lambda i: (i, 0)),
            pl.BlockSpec((1, HID), lambda i: (0, 0)),
            pl.BlockSpec((HID, HID), lambda i: (0, 0)),
        ],
        out_specs=pl.BlockSpec((BM, HID), lambda i: (i, 0)),
        out_shape=jax.ShapeDtypeStruct((NPAD, HID), F32),
    )(p, hs, dinv, b, W)


def _tc_gat_prep(p, hs, dinv, b, W3, att_src, att_dst):
    def body(p_ref, h_ref, d_ref, b_ref, w_ref, asr, adr, g_ref, as_ref, ad_ref):
        agg = p_ref[0] + p_ref[1] + h_ref[...]
        z = jnp.maximum(agg * d_ref[...] + b_ref[...], 0.0)
        h3 = jnp.dot(z, w_ref[...], preferred_element_type=F32)
        one = jnp.ones((BM, 1), F32)
        g_ref[...] = jnp.concatenate(
            [h3[:, 0:7], one, h3[:, 7:14], one], axis=1
        )
        va = asr[...]
        as0 = jnp.sum(h3[:, 0:7] * va[0:1, :], axis=1, keepdims=True)
        as1 = jnp.sum(h3[:, 7:14] * va[1:2, :], axis=1, keepdims=True)
        as_ref[...] = jnp.concatenate([as0, as1], axis=1)
        vd = adr[...]
        ad0 = jnp.sum(h3[:, 0:7] * vd[0:1, :], axis=1, keepdims=True)
        ad1 = jnp.sum(h3[:, 7:14] * vd[1:2, :], axis=1, keepdims=True)
        ad_ref[...] = jnp.concatenate([ad0, ad1], axis=1)

    return pl.pallas_call(
        body,
        grid=(NPAD // BM,),
        in_specs=[
            pl.BlockSpec((2, BM, HID), lambda i: (0, i, 0)),
            pl.BlockSpec((BM, HID), lambda i: (i, 0)),
            pl.BlockSpec((BM, 1), lambda i: (i, 0)),
            pl.BlockSpec((1, HID), lambda i: (0, 0)),
            pl.BlockSpec((HID, HEADS * NCLS), lambda i: (0, 0)),
            pl.BlockSpec((HEADS, NCLS), lambda i: (0, 0)),
            pl.BlockSpec((HEADS, NCLS), lambda i: (0, 0)),
        ],
        out_specs=[
            pl.BlockSpec((BM, GW), lambda i: (i, 0)),
            pl.BlockSpec((BM, 2), lambda i: (i, 0)),
            pl.BlockSpec((BM, 2), lambda i: (i, 0)),
        ],
        out_shape=[
            jax.ShapeDtypeStruct((N, GW), F32),
            jax.ShapeDtypeStruct((N, 2), F32),
            jax.ShapeDtypeStruct((N, 2), F32),
        ],
    )(p, hs, dinv, b, W3, att_src, att_dst)


def _tc_m(a_s, a_d):
    def body(as_ref, ad_ref, m_ref, am_ref):
        av = as_ref[...]
        amax = jnp.max(av, axis=0, keepdims=True)
        t = amax + ad_ref[...]
        m_ref[...] = jnp.maximum(t, 0.2 * t)
        am_ref[...] = jnp.broadcast_to(
            jnp.transpose(amax, (1, 0)), (HEADS, 16)
        )

    return pl.pallas_call(
        body,
        out_shape=[
            jax.ShapeDtypeStruct((N, 2), F32),
            jax.ShapeDtypeStruct((HEADS, 16), F32),
        ],
    )(a_s, a_d)


def _tc_final(pg, g, a_s, a_d, m, b3):
    def body(p_ref, g_ref, as_ref, ad_ref, m_ref, b_ref, o_ref):
        num = p_ref[0, :N, :] + p_ref[1, :N, :]
        es = as_ref[...] + ad_ref[...]
        es = jnp.maximum(es, 0.2 * es)
        ps = jnp.exp(es - m_ref[...])
        w = jnp.concatenate(
            [jnp.broadcast_to(ps[:, 0:1], (N, 8)),
             jnp.broadcast_to(ps[:, 1:2], (N, 8))], axis=1
        )
        num = num + g_ref[...] * w
        d0 = num[:, 7:8] + 1e-16
        d1 = num[:, 15:16] + 1e-16
        o = jnp.concatenate(
            [num[:, 0:7] / d0, num[:, 8:15] / d1], axis=1
        ) + b_ref[...]
        mx = jnp.max(o, axis=1, keepdims=True)
        t = o - mx
        o_ref[...] = t - jnp.log(jnp.sum(jnp.exp(t), axis=1, keepdims=True))

    return pl.pallas_call(
        body, out_shape=jax.ShapeDtypeStruct((N, HEADS * NCLS), F32)
    )(pg, g, a_s, a_d, m, b3)


def kernel(x, edge_index, W1, b1, W2, b2, W3, att_src, att_dst, b3):
    src2d = edge_index[0].reshape(NCHUNK, CH)
    dst2d = edge_index[1].reshape(NCHUNK, CH)

    degp = _sc_deg(dst2d).reshape(NWORK, NPAD)
    dinv = _tc_dinv(degp).reshape(NPAD, 1)

    hs1 = _tc_mm1(x, W1, dinv)
    p1 = _sc_prop(hs1, src2d, dst2d).reshape(2, NPAD, HID)
    hs2 = _tc_mid(p1, hs1, dinv, b1.reshape(1, HID), W2)
    p2 = _sc_prop(hs2, src2d, dst2d).reshape(2, NPAD, HID)
    g, a_s, a_d = _tc_gat_prep(
        p2, hs2, dinv, b2.reshape(1, HID), W3, att_src, att_dst
    )
    m, am = _tc_m(a_s, a_d)
    pg = _sc_gat(
        g, src2d, dst2d,
        a_s[:, 0], a_s[:, 1], a_d[:, 0], a_d[:, 1], am[0], am[1]
    ).reshape(2, NPAD, GW)
    return _tc_final(pg, g, a_s, a_d, m, b3.reshape(1, HEADS * NCLS))
